# Initial kernel scaffold; baseline (speedup 1.0000x reference)
import jax
import jax.numpy as jnp
from jax.experimental import pallas as pl


def kernel(user_embed, item_embed, kg_entity_embed, kg_relation_embed, trans_W, users, pos_items, neg_items, heads, relations, pos_tails, neg_tails):
    raise NotImplementedError("write your pallas kernel here")


# speedup vs baseline: 2.6489x; 2.6489x over previous
import jax
import jax.numpy as jnp
from jax.experimental import pallas as pl
from jax.experimental.pallas import tpu as pltpu

_EPS_SQ = 1e-24


def _l2n(x):
    return x * jax.lax.rsqrt(
        jnp.maximum(jnp.sum(x * x, axis=-1, keepdims=True), _EPS_SQ))


def _fused_body(R, rel_ref, h_ref, pt_ref, nt_ref, pi_ref, pkg_ref, ni_ref,
                nkg_ref, re_ref, w_ref, pc_out, pcb_out, nc_out, r_out,
                h_out, pt_out, nt_out):
    pc = pi_ref[...] + pkg_ref[...]
    pc_out[...] = pc
    pcb_out[...] = pc.astype(jnp.bfloat16)
    nc_out[...] = ni_ref[...] + nkg_ref[...]
    r_out[...] = _l2n(re_ref[...])

    rel = rel_ref[...]
    masks = [(rel == r).astype(jnp.bfloat16) for r in range(R)]

    def transr(e):
        exp = jnp.concatenate([e * m for m in masks], axis=1)
        proj = jnp.dot(exp, w_ref[...], preferred_element_type=jnp.float32)
        return _l2n(proj)

    h_out[...] = transr(h_ref[...])
    pt_out[...] = transr(pt_ref[...])
    nt_out[...] = transr(nt_ref[...])


def _pred_body(u_ref, p_ref, o_ref):
    o_ref[...] = jax.lax.dot_general(
        u_ref[...], p_ref[...], (((1,), (1,)), ((), ())),
        preferred_element_type=jnp.float32)


def _tile(n, target):
    t = target
    while t > 8 and n % t:
        t //= 2
    return t


def kernel(user_embed, item_embed, kg_entity_embed, kg_relation_embed,
           trans_W, users, pos_items, neg_items, heads, relations,
           pos_tails, neg_tails):
    B = int(users.shape[0])
    R, D, K = (int(s) for s in trans_W.shape)
    bf = jnp.bfloat16

    u_e = jnp.take(user_embed, users, axis=0)
    pos_i_e = jnp.take(item_embed, pos_items, axis=0)
    neg_i_e = jnp.take(item_embed, neg_items, axis=0)
    pos_i_kg_e = jnp.take(kg_entity_embed, pos_items, axis=0)
    neg_i_kg_e = jnp.take(kg_entity_embed, neg_items, axis=0)
    r_e = jnp.take(kg_relation_embed, relations, axis=0)
    h_e = jnp.take(kg_entity_embed, heads, axis=0).astype(bf)
    pt_e = jnp.take(kg_entity_embed, pos_tails, axis=0).astype(bf)
    nt_e = jnp.take(kg_entity_embed, neg_tails, axis=0).astype(bf)

    rel2 = jnp.clip(relations.astype(jnp.int32), 0, R - 1).reshape(B, 1)
    w_stack = trans_W.reshape(R * D, K).astype(bf)

    TB = _tile(B, 512)
    G = B // TB
    row_d = pl.BlockSpec((TB, D), lambda i: (i, 0))
    row_k = pl.BlockSpec((TB, K), lambda i: (i, 0))

    pos_comb, pos_comb_bf, neg_comb, r_o, h_o, pt_o, nt_o = pl.pallas_call(
        lambda *refs: _fused_body(R, *refs),
        grid=(G,),
        in_specs=[
            pl.BlockSpec((TB, 1), lambda i: (i, 0)),
            row_d, row_d, row_d,
            row_d, row_d, row_d, row_d,
            row_k,
            pl.BlockSpec((R * D, K), lambda i: (0, 0)),
        ],
        out_specs=(row_d, row_d, row_d, row_k, row_k, row_k, row_k),
        out_shape=(
            jax.ShapeDtypeStruct((B, D), jnp.float32),
            jax.ShapeDtypeStruct((B, D), bf),
            jax.ShapeDtypeStruct((B, D), jnp.float32),
            jax.ShapeDtypeStruct((B, K), jnp.float32),
            jax.ShapeDtypeStruct((B, K), jnp.float32),
            jax.ShapeDtypeStruct((B, K), jnp.float32),
            jax.ShapeDtypeStruct((B, K), jnp.float32),
        ),
        compiler_params=pltpu.CompilerParams(
            dimension_semantics=("parallel",)),
    )(rel2, h_e, pt_e, nt_e, pos_i_e, pos_i_kg_e, neg_i_e, neg_i_kg_e, r_e,
      w_stack)

    preds = pl.pallas_call(
        _pred_body,
        grid=(G,),
        in_specs=[
            pl.BlockSpec((TB, D), lambda i: (i, 0)),
            pl.BlockSpec((B, D), lambda i: (0, 0)),
        ],
        out_specs=pl.BlockSpec((TB, B), lambda i: (i, 0)),
        out_shape=jax.ShapeDtypeStruct((B, B), jnp.float32),
        compiler_params=pltpu.CompilerParams(
            dimension_semantics=("parallel",)),
    )(u_e.astype(bf), pos_comb_bf)

    return (u_e, pos_comb, neg_comb, h_o, r_o, pt_o, nt_o, preds)


# speedup vs baseline: 3.8493x; 1.4532x over previous
import jax
import jax.numpy as jnp
from jax.experimental import pallas as pl
from jax.experimental.pallas import tpu as pltpu

_EPS_SQ = 1e-24


def _l2n(x):
    return x * jax.lax.rsqrt(
        jnp.maximum(jnp.sum(x * x, axis=-1, keepdims=True), _EPS_SQ))


def _fused_body(R, rel_ref, ip_ref, in_ref, ep_ref, en_ref, h_ref, pt_ref,
                nt_ref, re_ref, w_ref, pc_out, pcb_out, nc_out, r_out,
                h_out, pt_out, nt_out):
    pc = ip_ref[...] + ep_ref[...]
    pc_out[...] = pc
    pcb_out[...] = pc.astype(jnp.bfloat16)
    nc_out[...] = in_ref[...] + en_ref[...]
    r_out[...] = _l2n(re_ref[...])

    rel = rel_ref[...]
    masks = [(rel == r).astype(jnp.bfloat16) for r in range(R)]

    def transr(e_ref):
        e = e_ref[...].astype(jnp.bfloat16)
        exp = jnp.concatenate([e * m for m in masks], axis=1)
        proj = jnp.dot(exp, w_ref[...], preferred_element_type=jnp.float32)
        return _l2n(proj)

    h_out[...] = transr(h_ref)
    pt_out[...] = transr(pt_ref)
    nt_out[...] = transr(nt_ref)


def _pred_body(u_ref, p_ref, o_ref):
    o_ref[...] = jax.lax.dot_general(
        u_ref[...].astype(jnp.bfloat16), p_ref[...],
        (((1,), (1,)), ((), ())), preferred_element_type=jnp.float32)


def _tile(n, target):
    t = target
    while t > 8 and n % t:
        t //= 2
    return t


def kernel(user_embed, item_embed, kg_entity_embed, kg_relation_embed,
           trans_W, users, pos_items, neg_items, heads, relations,
           pos_tails, neg_tails):
    B = int(users.shape[0])
    R, D, K = (int(s) for s in trans_W.shape)
    bf = jnp.bfloat16

    u_e = jnp.take(user_embed, users, axis=0)
    item_rows = jnp.take(item_embed,
                         jnp.concatenate([pos_items, neg_items]), axis=0)
    ent_rows = jnp.take(
        kg_entity_embed,
        jnp.concatenate([pos_items, neg_items, heads, pos_tails, neg_tails]),
        axis=0)
    r_e = jnp.take(kg_relation_embed, relations, axis=0)

    rel2 = jnp.clip(relations.astype(jnp.int32), 0, R - 1).reshape(B, 1)
    w_stack = trans_W.reshape(R * D, K).astype(bf)

    TB = _tile(B, 512)
    G = B // TB
    row_d = pl.BlockSpec((TB, D), lambda i: (i, 0))
    row_k = pl.BlockSpec((TB, K), lambda i: (i, 0))

    def seg(j):
        return pl.BlockSpec((TB, D), lambda i, j=j: (i + j * G, 0))

    pos_comb, pos_comb_bf, neg_comb, r_o, h_o, pt_o, nt_o = pl.pallas_call(
        lambda *refs: _fused_body(R, *refs),
        grid=(G,),
        in_specs=[
            pl.BlockSpec((TB, 1), lambda i: (i, 0)),
            seg(0), seg(1),
            seg(0), seg(1), seg(2), seg(3), seg(4),
            row_k,
            pl.BlockSpec((R * D, K), lambda i: (0, 0)),
        ],
        out_specs=(row_d, row_d, row_d, row_k, row_k, row_k, row_k),
        out_shape=(
            jax.ShapeDtypeStruct((B, D), jnp.float32),
            jax.ShapeDtypeStruct((B, D), bf),
            jax.ShapeDtypeStruct((B, D), jnp.float32),
            jax.ShapeDtypeStruct((B, K), jnp.float32),
            jax.ShapeDtypeStruct((B, K), jnp.float32),
            jax.ShapeDtypeStruct((B, K), jnp.float32),
            jax.ShapeDtypeStruct((B, K), jnp.float32),
        ),
        compiler_params=pltpu.CompilerParams(
            dimension_semantics=("parallel",)),
    )(rel2, item_rows, item_rows, ent_rows, ent_rows, ent_rows, ent_rows,
      ent_rows, r_e, w_stack)

    preds = pl.pallas_call(
        _pred_body,
        grid=(G,),
        in_specs=[
            pl.BlockSpec((TB, D), lambda i: (i, 0)),
            pl.BlockSpec((B, D), lambda i: (0, 0)),
        ],
        out_specs=pl.BlockSpec((TB, B), lambda i: (i, 0)),
        out_shape=jax.ShapeDtypeStruct((B, B), jnp.float32),
        compiler_params=pltpu.CompilerParams(
            dimension_semantics=("parallel",)),
    )(u_e, pos_comb_bf)

    return (u_e, pos_comb, neg_comb, h_o, r_o, pt_o, nt_o, preds)


# speedup vs baseline: 5.5119x; 1.4319x over previous
import jax
import jax.numpy as jnp
from jax.experimental import pallas as pl
from jax.experimental.pallas import tpu as pltpu

_EPS_SQ = 1e-24


def _l2n(x):
    return x * jax.lax.rsqrt(
        jnp.maximum(jnp.sum(x * x, axis=-1, keepdims=True), _EPS_SQ))


def _fused_body(R, rel_ref, ip_ref, in_ref, ep_ref, en_ref, h_ref, pt_ref,
                nt_ref, re_ref, w_ref, pc_out, pcb_out, nc_out, r_out,
                h_out, pt_out, nt_out):
    pc = ip_ref[...] + ep_ref[...]
    pc_out[...] = pc
    pcb_out[...] = pc.astype(jnp.bfloat16)
    nc_out[...] = in_ref[...] + en_ref[...]
    r_out[...] = _l2n(re_ref[...])

    rel = rel_ref[...]
    masks = [(rel == r).astype(jnp.bfloat16) for r in range(R)]

    def transr(e_ref):
        e = e_ref[...].astype(jnp.bfloat16)
        exp = jnp.concatenate([e * m for m in masks], axis=1)
        proj = jnp.dot(exp, w_ref[...], preferred_element_type=jnp.float32)
        return _l2n(proj)

    h_out[...] = transr(h_ref)
    pt_out[...] = transr(pt_ref)
    nt_out[...] = transr(nt_ref)


def _pred_body(u_ref, p_ref, o_ref):
    o_ref[...] = jax.lax.dot_general(
        u_ref[...].astype(jnp.bfloat16), p_ref[...],
        (((1,), (1,)), ((), ())), preferred_element_type=jnp.float32)


def _tile(n, target):
    t = target
    while t > 8 and n % t:
        t //= 2
    return t


def kernel(user_embed, item_embed, kg_entity_embed, kg_relation_embed,
           trans_W, users, pos_items, neg_items, heads, relations,
           pos_tails, neg_tails):
    B = int(users.shape[0])
    R, D, K = (int(s) for s in trans_W.shape)
    bf = jnp.bfloat16

    u_e = jnp.take(user_embed, users, axis=0, mode="clip")
    item_rows = jnp.take(item_embed,
                         jnp.concatenate([pos_items, neg_items]), axis=0,
                         mode="clip")
    ent_rows = jnp.take(
        kg_entity_embed,
        jnp.concatenate([pos_items, neg_items, heads, pos_tails, neg_tails]),
        axis=0, mode="clip")
    r_e = jnp.take(kg_relation_embed, relations, axis=0, mode="clip")

    rel2 = jnp.clip(relations.astype(jnp.int32), 0, R - 1).reshape(B, 1)
    w_stack = trans_W.reshape(R * D, K).astype(bf)

    TB = _tile(B, 512)
    G = B // TB
    row_d = pl.BlockSpec((TB, D), lambda i: (i, 0))
    row_k = pl.BlockSpec((TB, K), lambda i: (i, 0))

    def seg(j):
        return pl.BlockSpec((TB, D), lambda i, j=j: (i + j * G, 0))

    pos_comb, pos_comb_bf, neg_comb, r_o, h_o, pt_o, nt_o = pl.pallas_call(
        lambda *refs: _fused_body(R, *refs),
        grid=(G,),
        in_specs=[
            pl.BlockSpec((TB, 1), lambda i: (i, 0)),
            seg(0), seg(1),
            seg(0), seg(1), seg(2), seg(3), seg(4),
            row_k,
            pl.BlockSpec((R * D, K), lambda i: (0, 0)),
        ],
        out_specs=(row_d, row_d, row_d, row_k, row_k, row_k, row_k),
        out_shape=(
            jax.ShapeDtypeStruct((B, D), jnp.float32),
            jax.ShapeDtypeStruct((B, D), bf),
            jax.ShapeDtypeStruct((B, D), jnp.float32),
            jax.ShapeDtypeStruct((B, K), jnp.float32),
            jax.ShapeDtypeStruct((B, K), jnp.float32),
            jax.ShapeDtypeStruct((B, K), jnp.float32),
            jax.ShapeDtypeStruct((B, K), jnp.float32),
        ),
        compiler_params=pltpu.CompilerParams(
            dimension_semantics=("parallel",)),
    )(rel2, item_rows, item_rows, ent_rows, ent_rows, ent_rows, ent_rows,
      ent_rows, r_e, w_stack)

    preds = pl.pallas_call(
        _pred_body,
        grid=(G,),
        in_specs=[
            pl.BlockSpec((TB, D), lambda i: (i, 0)),
            pl.BlockSpec((B, D), lambda i: (0, 0)),
        ],
        out_specs=pl.BlockSpec((TB, B), lambda i: (i, 0)),
        out_shape=jax.ShapeDtypeStruct((B, B), jnp.float32),
        compiler_params=pltpu.CompilerParams(
            dimension_semantics=("parallel",)),
    )(u_e, pos_comb_bf)

    return (u_e, pos_comb, neg_comb, h_o, r_o, pt_o, nt_o, preds)


# speedup vs baseline: 5.5410x; 1.0053x over previous
import jax
import jax.numpy as jnp
from jax.experimental import pallas as pl
from jax.experimental.pallas import tpu as pltpu

_EPS_SQ = 1e-24


def _l2n(x):
    return x * jax.lax.rsqrt(
        jnp.maximum(jnp.sum(x * x, axis=-1, keepdims=True), _EPS_SQ))


def _fused_body(R, oh_ref, ip_ref, in_ref, ep_ref, en_ref, h_ref, pt_ref,
                nt_ref, relt_ref, w_ref, pc_out, pcb_out, nc_out, r_out,
                h_out, pt_out, nt_out):
    pc = ip_ref[...] + ep_ref[...]
    pc_out[...] = pc
    pcb_out[...] = pc.astype(jnp.bfloat16)
    nc_out[...] = in_ref[...] + en_ref[...]

    oh = oh_ref[...].astype(jnp.bfloat16)
    r_out[...] = jnp.dot(oh, _l2n(relt_ref[...]).astype(jnp.bfloat16),
                         preferred_element_type=jnp.float32)

    masks = [oh[:, r:r + 1] for r in range(R)]

    def transr(e_ref):
        e = e_ref[...].astype(jnp.bfloat16)
        exp = jnp.concatenate([e * m for m in masks], axis=1)
        proj = jnp.dot(exp, w_ref[...], preferred_element_type=jnp.float32)
        return _l2n(proj)

    h_out[...] = transr(h_ref)
    pt_out[...] = transr(pt_ref)
    nt_out[...] = transr(nt_ref)


def _pred_body(u_ref, p_ref, o_ref):
    o_ref[...] = jax.lax.dot_general(
        u_ref[...].astype(jnp.bfloat16), p_ref[...],
        (((1,), (1,)), ((), ())), preferred_element_type=jnp.float32)


def _tile(n, target):
    t = target
    while t > 8 and n % t:
        t //= 2
    return t


def kernel(user_embed, item_embed, kg_entity_embed, kg_relation_embed,
           trans_W, users, pos_items, neg_items, heads, relations,
           pos_tails, neg_tails):
    B = int(users.shape[0])
    R, D, K = (int(s) for s in trans_W.shape)
    bf = jnp.bfloat16

    u_e = jnp.take(user_embed, users, axis=0, mode="clip")
    item_rows = jnp.take(item_embed,
                         jnp.concatenate([pos_items, neg_items]), axis=0,
                         mode="clip")
    ent_rows = jnp.take(
        kg_entity_embed,
        jnp.concatenate([pos_items, neg_items, heads, pos_tails, neg_tails]),
        axis=0, mode="clip")

    onehot = (relations.astype(jnp.int32)[:, None]
              == jnp.arange(R, dtype=jnp.int32)[None, :]).astype(jnp.float32)
    w_stack = trans_W.reshape(R * D, K).astype(bf)

    TB = _tile(B, 512)
    G = B // TB
    row_d = pl.BlockSpec((TB, D), lambda i: (i, 0))
    row_k = pl.BlockSpec((TB, K), lambda i: (i, 0))

    def seg(j):
        return pl.BlockSpec((TB, D), lambda i, j=j: (i + j * G, 0))

    pos_comb, pos_comb_bf, neg_comb, r_o, h_o, pt_o, nt_o = pl.pallas_call(
        lambda *refs: _fused_body(R, *refs),
        grid=(G,),
        in_specs=[
            pl.BlockSpec((TB, R), lambda i: (i, 0)),
            seg(0), seg(1),
            seg(0), seg(1), seg(2), seg(3), seg(4),
            pl.BlockSpec((R, K), lambda i: (0, 0)),
            pl.BlockSpec((R * D, K), lambda i: (0, 0)),
        ],
        out_specs=(row_d, row_d, row_d, row_k, row_k, row_k, row_k),
        out_shape=(
            jax.ShapeDtypeStruct((B, D), jnp.float32),
            jax.ShapeDtypeStruct((B, D), bf),
            jax.ShapeDtypeStruct((B, D), jnp.float32),
            jax.ShapeDtypeStruct((B, K), jnp.float32),
            jax.ShapeDtypeStruct((B, K), jnp.float32),
            jax.ShapeDtypeStruct((B, K), jnp.float32),
            jax.ShapeDtypeStruct((B, K), jnp.float32),
        ),
        compiler_params=pltpu.CompilerParams(
            dimension_semantics=("parallel",)),
    )(onehot, item_rows, item_rows, ent_rows, ent_rows, ent_rows, ent_rows,
      ent_rows, kg_relation_embed, w_stack)

    preds = pl.pallas_call(
        _pred_body,
        grid=(G,),
        in_specs=[
            pl.BlockSpec((TB, D), lambda i: (i, 0)),
            pl.BlockSpec((B, D), lambda i: (0, 0)),
        ],
        out_specs=pl.BlockSpec((TB, B), lambda i: (i, 0)),
        out_shape=jax.ShapeDtypeStruct((B, B), jnp.float32),
        compiler_params=pltpu.CompilerParams(
            dimension_semantics=("parallel",)),
    )(u_e, pos_comb_bf)

    return (u_e, pos_comb, neg_comb, h_o, r_o, pt_o, nt_o, preds)
